# Initial kernel scaffold; baseline (speedup 1.0000x reference)
import jax
import jax.numpy as jnp
from jax.experimental import pallas as pl
from jax.experimental.pallas import tpu as pltpu

N_MEM = 4
D = 768
B = 4
L = 8192
TOPK = 64
TILE = 1024
NT = L // TILE

_CONTRACT_LAST = (((1,), (1,)), ((), ()))


def _slope(l):
    return 0.01 + (0.2 / N_MEM) * l.astype(jnp.float32)


def _logits_kernel(q_ref, w1_ref, b1_ref, w2_ref, b2_ref, gw_ref, out_ref):
    l = pl.program_id(0)
    t = pl.program_id(2)
    q = q_ref[0]
    h = jax.lax.dot_general(q, w1_ref[0], _CONTRACT_LAST,
                            preferred_element_type=jnp.float32)
    h = h + b1_ref[0]
    h = jnp.where(h > 0, h, _slope(l) * h)
    h = jax.lax.dot_general(h, w2_ref[0], _CONTRACT_LAST,
                            preferred_element_type=jnp.float32)
    x = jnp.tanh(h + b2_ref[0])
    lg = jax.lax.dot_general(gw_ref[0], x, _CONTRACT_LAST,
                             preferred_element_type=jnp.float32)
    out_ref[0, :, pl.ds(t * TILE, TILE)] = lg


def _topk_kernel(lg_ref, idx_ref, w_ref, ms_ref):
    lr = lg_ref[...]
    m = jnp.max(lr, axis=1, keepdims=True)
    s = jnp.sum(jnp.exp(lr - m), axis=1, keepdims=True)
    iota = jax.lax.broadcasted_iota(jnp.int32, lr.shape, 1)
    lane = jax.lax.broadcasted_iota(jnp.int32, (lr.shape[0], TOPK), 1)
    ms_ref[...] = lr

    def body(i, carry):
        idxs, wts = carry
        masked = ms_ref[...]
        cur = jnp.max(masked, axis=1, keepdims=True)
        pos = jnp.min(jnp.where(masked == cur, iota, L), axis=1, keepdims=True)
        w = jnp.exp(cur - m) / s
        idxs = jnp.where(lane == i, pos, idxs)
        wts = jnp.where(lane == i, w, wts)
        ms_ref[...] = jnp.where(iota == pos, -jnp.inf, masked)
        return idxs, wts

    idxs, wts = jax.lax.fori_loop(
        0, TOPK, body,
        (jnp.zeros((lr.shape[0], TOPK), jnp.int32),
         jnp.zeros((lr.shape[0], TOPK), jnp.float32)))
    idx_ref[...] = idxs
    w_ref[...] = wts


def _combine_kernel(idx_ref, w_ref, q_hbm, w1_ref, b1_ref, w2_ref, b2_ref,
                    out_ref, gq, sem):
    l = pl.program_id(0)
    b = pl.program_id(1)

    def start(k, _):
        pltpu.make_async_copy(q_hbm.at[b, idx_ref[0, 0, k]], gq.at[k],
                              sem).start()
        return 0

    jax.lax.fori_loop(0, TOPK, start, 0)

    def wait(k, _):
        pltpu.make_async_copy(q_hbm.at[b, 0], gq.at[k], sem).wait()
        return 0

    jax.lax.fori_loop(0, TOPK, wait, 0)

    q = gq[...]
    h = jax.lax.dot_general(q, w1_ref[0], _CONTRACT_LAST,
                            preferred_element_type=jnp.float32)
    h = h + b1_ref[0]
    h = jnp.where(h > 0, h, _slope(l) * h)
    h = jax.lax.dot_general(h, w2_ref[0], _CONTRACT_LAST,
                            preferred_element_type=jnp.float32)
    x = jnp.tanh(h + b2_ref[0])
    o = jax.lax.dot_general(w_ref[0], x, (((1,), (0,)), ((), ())),
                            preferred_element_type=jnp.float32)
    n = jnp.sqrt(jnp.sum(o * o))
    out_ref[0, 0] = o / jnp.maximum(n, 1e-12)


@jax.jit
def kernel(query, mem_W1, mem_b1, mem_W2, mem_b2, gate_W, gate_b):
    del gate_b
    b1 = mem_b1.reshape(N_MEM, 1, D)
    b2 = mem_b2.reshape(N_MEM, 1, D)

    logits = pl.pallas_call(
        _logits_kernel,
        grid=(N_MEM, B, NT),
        in_specs=[
            pl.BlockSpec((1, TILE, D), lambda l, b, t: (b, t, 0)),
            pl.BlockSpec((1, D, D), lambda l, b, t: (l, 0, 0)),
            pl.BlockSpec((1, 1, D), lambda l, b, t: (l, 0, 0)),
            pl.BlockSpec((1, D, D), lambda l, b, t: (l, 0, 0)),
            pl.BlockSpec((1, 1, D), lambda l, b, t: (l, 0, 0)),
            pl.BlockSpec((1, 1, D), lambda l, b, t: (l, 0, 0)),
        ],
        out_specs=pl.BlockSpec((1, 1, L), lambda l, b, t: (l * B + b, 0, 0)),
        out_shape=jax.ShapeDtypeStruct((N_MEM * B, 1, L), jnp.float32),
    )(query, mem_W1, b1, mem_W2, b2, gate_W)

    idxs, wts = pl.pallas_call(
        _topk_kernel,
        out_shape=[
            jax.ShapeDtypeStruct((N_MEM * B, TOPK), jnp.int32),
            jax.ShapeDtypeStruct((N_MEM * B, TOPK), jnp.float32),
        ],
        scratch_shapes=[pltpu.VMEM((N_MEM * B, L), jnp.float32)],
    )(logits.reshape(N_MEM * B, L))

    out = pl.pallas_call(
        _combine_kernel,
        grid=(N_MEM, B),
        in_specs=[
            pl.BlockSpec((1, 1, TOPK), lambda l, b: (l * B + b, 0, 0),
                         memory_space=pltpu.SMEM),
            pl.BlockSpec((1, 1, TOPK), lambda l, b: (l * B + b, 0, 0)),
            pl.BlockSpec(memory_space=pltpu.ANY),
            pl.BlockSpec((1, D, D), lambda l, b: (l, 0, 0)),
            pl.BlockSpec((1, 1, D), lambda l, b: (l, 0, 0)),
            pl.BlockSpec((1, D, D), lambda l, b: (l, 0, 0)),
            pl.BlockSpec((1, 1, D), lambda l, b: (l, 0, 0)),
        ],
        out_specs=pl.BlockSpec((1, 1, 1, D), lambda l, b: (b, l, 0, 0)),
        out_shape=jax.ShapeDtypeStruct((B, N_MEM, 1, D), jnp.float32),
        scratch_shapes=[
            pltpu.VMEM((TOPK, D), jnp.float32),
            pltpu.SemaphoreType.DMA,
        ],
    )(idxs.reshape(N_MEM * B, 1, TOPK), wts.reshape(N_MEM * B, 1, TOPK),
      query, mem_W1, b1, mem_W2, b2)

    return out.reshape(B, N_MEM, D)


# speedup vs baseline: 1.7293x; 1.7293x over previous
import jax
import jax.numpy as jnp
from jax.experimental import pallas as pl
from jax.experimental.pallas import tpu as pltpu

N_MEM = 4
D = 768
B = 4
L = 8192
TOPK = 64
TILE = 1024
NT = L // TILE

_CONTRACT_LAST = (((1,), (1,)), ((), ()))


def _slope(l):
    return 0.01 + (0.2 / N_MEM) * l.astype(jnp.float32)


def _logits_kernel(q_ref, w1_ref, b1_ref, w2_ref, b2_ref, gw_ref, out_ref):
    l = pl.program_id(0)
    t = pl.program_id(2)
    q = q_ref[0]
    h = jax.lax.dot_general(q, w1_ref[0], _CONTRACT_LAST,
                            preferred_element_type=jnp.float32)
    h = h + b1_ref[0]
    h = jnp.where(h > 0, h, _slope(l) * h)
    h = jax.lax.dot_general(h, w2_ref[0], _CONTRACT_LAST,
                            preferred_element_type=jnp.float32)
    x = jnp.tanh(h + b2_ref[0])
    lg = jax.lax.dot_general(gw_ref[0], x, _CONTRACT_LAST,
                             preferred_element_type=jnp.float32)
    out_ref[0, :, pl.ds(t * TILE, TILE)] = lg


def _topk_kernel(lg_ref, idx_ref, w_ref, ms_ref):
    lr = lg_ref[...]
    m = jnp.max(lr, axis=1, keepdims=True)
    s = jnp.sum(jnp.exp(lr - m), axis=1, keepdims=True)
    iota = jax.lax.broadcasted_iota(jnp.int32, lr.shape, 1)
    lane = jax.lax.broadcasted_iota(jnp.int32, (lr.shape[0], TOPK), 1)
    ms_ref[...] = lr

    def body(i, carry):
        idxs, wts = carry
        masked = ms_ref[...]
        cur = jnp.max(masked, axis=1, keepdims=True)
        pos = jnp.min(jnp.where(masked == cur, iota, L), axis=1, keepdims=True)
        w = jnp.exp(cur - m) / s
        idxs = jnp.where(lane == i, pos, idxs)
        wts = jnp.where(lane == i, w, wts)
        ms_ref[...] = jnp.where(iota == pos, -jnp.inf, masked)
        return idxs, wts

    idxs, wts = jax.lax.fori_loop(
        0, TOPK, body,
        (jnp.zeros((lr.shape[0], TOPK), jnp.int32),
         jnp.zeros((lr.shape[0], TOPK), jnp.float32)))
    idx_ref[...] = idxs
    w_ref[...] = wts


def _combine_kernel(idx_ref, w_ref, q_hbm, w1_ref, b1_ref, w2_ref, b2_ref,
                    out_ref, gq, sem):
    l = pl.program_id(0)
    b = pl.program_id(1)

    def start(k, _):
        pltpu.make_async_copy(q_hbm.at[b, idx_ref[0, 0, k]], gq.at[k],
                              sem).start()
        return 0

    jax.lax.fori_loop(0, TOPK, start, 0)

    def wait(k, _):
        pltpu.make_async_copy(q_hbm.at[b, 0], gq.at[k], sem).wait()
        return 0

    jax.lax.fori_loop(0, TOPK, wait, 0)

    q = gq[...]
    h = jax.lax.dot_general(q, w1_ref[0], _CONTRACT_LAST,
                            preferred_element_type=jnp.float32)
    h = h + b1_ref[0]
    h = jnp.where(h > 0, h, _slope(l) * h)
    h = jax.lax.dot_general(h, w2_ref[0], _CONTRACT_LAST,
                            preferred_element_type=jnp.float32)
    x = jnp.tanh(h + b2_ref[0])
    o = jax.lax.dot_general(w_ref[0], x, (((1,), (0,)), ((), ())),
                            preferred_element_type=jnp.float32)
    n = jnp.sqrt(jnp.sum(o * o))
    out_ref[0, 0] = o / jnp.maximum(n, 1e-12)


@jax.jit
def kernel(query, mem_W1, mem_b1, mem_W2, mem_b2, gate_W, gate_b):
    del gate_b
    b1 = mem_b1.reshape(N_MEM, 1, D)
    b2 = mem_b2.reshape(N_MEM, 1, D)

    logits = pl.pallas_call(
        _logits_kernel,
        grid=(N_MEM, B, NT),
        in_specs=[
            pl.BlockSpec((1, TILE, D), lambda l, b, t: (b, t, 0)),
            pl.BlockSpec((1, D, D), lambda l, b, t: (l, 0, 0)),
            pl.BlockSpec((1, 1, D), lambda l, b, t: (l, 0, 0)),
            pl.BlockSpec((1, D, D), lambda l, b, t: (l, 0, 0)),
            pl.BlockSpec((1, 1, D), lambda l, b, t: (l, 0, 0)),
            pl.BlockSpec((1, 1, D), lambda l, b, t: (l, 0, 0)),
        ],
        out_specs=pl.BlockSpec((1, 1, L), lambda l, b, t: (l * B + b, 0, 0)),
        out_shape=jax.ShapeDtypeStruct((N_MEM * B, 1, L), jnp.float32),
    )(query, mem_W1, b1, mem_W2, b2, gate_W)

    idxs, wts = pl.pallas_call(
        _topk_kernel,
        out_shape=[
            jax.ShapeDtypeStruct((N_MEM * B, TOPK), jnp.int32),
            jax.ShapeDtypeStruct((N_MEM * B, TOPK), jnp.float32),
        ],
        scratch_shapes=[pltpu.VMEM((N_MEM * B, L), jnp.float32)],
    )(logits.reshape(N_MEM * B, L))

    out = pl.pallas_call(
        _combine_kernel,
        grid=(N_MEM, B),
        in_specs=[
            pl.BlockSpec((1, 1, TOPK), lambda l, b: (l * B + b, 0, 0),
                         memory_space=pltpu.SMEM),
            pl.BlockSpec((1, 1, TOPK), lambda l, b: (l * B + b, 0, 0)),
            pl.BlockSpec(memory_space=pl.ANY),
            pl.BlockSpec((1, D, D), lambda l, b: (l, 0, 0)),
            pl.BlockSpec((1, 1, D), lambda l, b: (l, 0, 0)),
            pl.BlockSpec((1, D, D), lambda l, b: (l, 0, 0)),
            pl.BlockSpec((1, 1, D), lambda l, b: (l, 0, 0)),
        ],
        out_specs=pl.BlockSpec((1, 1, 1, D), lambda l, b: (b, l, 0, 0)),
        out_shape=jax.ShapeDtypeStruct((B, N_MEM, 1, D), jnp.float32),
        scratch_shapes=[
            pltpu.VMEM((TOPK, D), jnp.float32),
            pltpu.SemaphoreType.DMA,
        ],
    )(idxs.reshape(N_MEM * B, 1, TOPK), wts.reshape(N_MEM * B, 1, TOPK),
      query, mem_W1, b1, mem_W2, b2)

    return out.reshape(B, N_MEM, D)
